# baseline (speedup 1.0000x reference)
import jax
import jax.numpy as jnp
from jax.experimental import pallas as pl
from jax.experimental.pallas import tpu as pltpu


def _copy_kernel(prev_ref, out_ref):
    out_ref[...] = prev_ref[...]


def _scatter_kernel(idx_ref, cur_ref, base_ref, out_ref):
    del idx_ref, base_ref
    out_ref[...] = cur_ref[...]


def kernel(prev, cur, dim, idx, inp_seq_len):
    B, H, KV, D = prev.shape
    Q = cur.shape[2]
    idx = (idx + (jnp.asarray(dim, dtype=idx.dtype) - 2)).astype(jnp.int32)

    prev3 = prev.reshape(B * H, KV, D)
    cur3 = cur.reshape(B * H, Q, D)

    copied = pl.pallas_call(
        _copy_kernel,
        grid=(B * H,),
        in_specs=[pl.BlockSpec((1, KV, D), lambda i: (i, 0, 0))],
        out_specs=pl.BlockSpec((1, KV, D), lambda i: (i, 0, 0)),
        out_shape=jax.ShapeDtypeStruct((B * H, KV, D), prev.dtype),
        compiler_params=pltpu.CompilerParams(dimension_semantics=("parallel",)),
    )(prev3)

    grid_spec = pltpu.PrefetchScalarGridSpec(
        num_scalar_prefetch=1,
        grid=(1,),
        in_specs=[
            pl.BlockSpec((B * H, Q, D), lambda i, idx_ref: (0, 0, 0)),
            pl.BlockSpec(memory_space=pl.ANY),
        ],
        out_specs=pl.BlockSpec(
            (B * H, Q, D), lambda i, idx_ref: (0, idx_ref[0] // Q, 0)
        ),
    )
    out3 = pl.pallas_call(
        _scatter_kernel,
        grid_spec=grid_spec,
        out_shape=jax.ShapeDtypeStruct((B * H, KV, D), prev.dtype),
        input_output_aliases={2: 0},
    )(idx, cur3, copied)
    return out3.reshape(B, H, KV, D)
